# baseline (speedup 1.0000x reference)
import functools

import jax
import jax.numpy as jnp
from jax import lax
from jax.experimental import pallas as pl
from jax.experimental.pallas import tpu as pltpu
from jax.experimental.pallas import tpu_sc as plsc

_NC = 2
_NS = 16
_NW = _NC * _NS


def _stage1(x_ref, w1_ref, b1_ref, a_ref, b_ref):
    x = x_ref[...]
    w = w1_ref[...]
    c = x.shape[1]
    a_ref[...] = jnp.dot(x, w[:c, :], preferred_element_type=jnp.float32).astype(
        jnp.bfloat16
    )
    b_ref[...] = jnp.dot(x, w[c:, :], preferred_element_type=jnp.float32) + b1_ref[...]


def _make_gather(n, h, e, chunk, nbuf):
    epw = e // _NW
    nchunk = epw // chunk
    mesh = plsc.VectorSubcoreMesh(
        core_axis_name="c", subcore_axis_name="s", num_cores=_NC, num_subcores=_NS
    )

    @functools.partial(
        pl.kernel,
        out_type=jax.ShapeDtypeStruct((e, h), jnp.bfloat16),
        mesh=mesh,
        scratch_types=[
            pltpu.VMEM((epw,), jnp.int32),
            pltpu.VMEM_SHARED((n, h), jnp.bfloat16),
            [pltpu.VMEM((chunk, h), jnp.bfloat16) for _ in range(nbuf)],
            [pltpu.SemaphoreType.DMA for _ in range(nbuf)],
            [pltpu.SemaphoreType.DMA for _ in range(nbuf)],
        ],
        compiler_params=pltpu.CompilerParams(use_tc_tiling_on_sc=False),
    )
    def gather_kernel(a_hbm, idx_hbm, out_hbm, idx_all, a_sh, bufs, gsems, ssems):
        wid = lax.axis_index("s") * _NC + lax.axis_index("c")
        base = wid * epw

        @pl.when(lax.axis_index("s") == 0)
        def _():
            pltpu.sync_copy(a_hbm, a_sh)

        pltpu.sync_copy(idx_hbm.at[pl.ds(base, epw)], idx_all)
        plsc.subcore_barrier()

        def gather_cp(c, b):
            return pltpu.make_async_copy(
                a_sh.at[idx_all.at[pl.ds(c * chunk, chunk)]], bufs[b], gsems[b]
            )

        def store_cp(c, b):
            return pltpu.make_async_copy(
                bufs[b], out_hbm.at[pl.ds(base + c * chunk, chunk)], ssems[b]
            )

        for c in range(min(nbuf, nchunk)):
            gather_cp(c, c % nbuf).start()
        for c in range(nchunk):
            b = c % nbuf
            gather_cp(c, b).wait()
            store_cp(c, b).start()
            if c + nbuf < nchunk:
                store_cp(c, b).wait()
                gather_cp(c + nbuf, b).start()
            else:
                store_cp(c, b).wait()

    return gather_kernel


def _stage3(k):
    def body(rep_ref, b_ref, w2t_ref, b2_ref, o_ref):
        z = rep_ref[...].astype(jnp.float32)
        g = b_ref.shape[0]
        z3 = z.reshape(g, k, z.shape[1]) + b_ref[...][:, None, :]
        hh = z3 * 0.5 * (1.0 + lax.erf(z3 * (2.0**-0.5)))
        t = jnp.sum(hh, axis=1)
        o_ref[...] = (
            jnp.dot(t, w2t_ref[...], preferred_element_type=jnp.float32)
            + b2_ref[...]
        )

    return body


def kernel(in_features, neighbors_index, neighbors_row_splits, W1, b1, W2, b2):
    n, c = in_features.shape
    e = neighbors_index.shape[0]
    h = W1.shape[1]
    co = W2.shape[1]
    k = e // n

    a_tab, b_tab = pl.pallas_call(
        _stage1,
        out_shape=[
            jax.ShapeDtypeStruct((n, h), jnp.bfloat16),
            jax.ShapeDtypeStruct((n, h), jnp.float32),
        ],
    )(in_features, W1, b1.reshape(1, h))

    rep = _make_gather(n, h, e, 1000, 4)(a_tab, neighbors_index)

    w2t = W2 * (1.0 / k)

    bn = 400
    out = pl.pallas_call(
        _stage3(k),
        grid=(n // bn,),
        in_specs=[
            pl.BlockSpec((bn * k, h), lambda i: (i, 0)),
            pl.BlockSpec((bn, h), lambda i: (i, 0)),
            pl.BlockSpec((h, co), lambda i: (0, 0)),
            pl.BlockSpec((1, co), lambda i: (0, 0)),
        ],
        out_specs=pl.BlockSpec((bn, co), lambda i: (i, 0)),
        out_shape=jax.ShapeDtypeStruct((n, co), jnp.float32),
    )(rep, b_tab, w2t, b2.reshape(1, co))

    return out


# speedup vs baseline: 1.8653x; 1.8653x over previous
import functools

import jax
import jax.numpy as jnp
from jax import lax
from jax.experimental import pallas as pl
from jax.experimental.pallas import tpu as pltpu
from jax.experimental.pallas import tpu_sc as plsc

_NC = 2
_NS = 16
_NW = _NC * _NS


def _stage1(x_ref, w1_ref, b1_ref, a_ref, b_ref):
    x = x_ref[...]
    w = w1_ref[...]
    c = x.shape[1]
    a_ref[...] = jnp.dot(x, w[:c, :], preferred_element_type=jnp.float32).astype(
        jnp.bfloat16
    )
    b_ref[...] = jnp.dot(x, w[c:, :], preferred_element_type=jnp.float32) + b1_ref[...]


def _make_gather(n, h, e, chunk, nbuf):
    epw = e // _NW
    nchunk = epw // chunk
    mesh = plsc.VectorSubcoreMesh(
        core_axis_name="c", subcore_axis_name="s", num_cores=_NC, num_subcores=_NS
    )

    @functools.partial(
        pl.kernel,
        out_type=jax.ShapeDtypeStruct((e, h), jnp.bfloat16),
        mesh=mesh,
        scratch_types=[
            pltpu.VMEM((epw,), jnp.int32),
            pltpu.VMEM_SHARED((n, h), jnp.bfloat16),
            [pltpu.VMEM((chunk, h), jnp.bfloat16) for _ in range(nbuf)],
            [pltpu.SemaphoreType.DMA for _ in range(nbuf)],
            [pltpu.SemaphoreType.DMA for _ in range(nbuf)],
        ],
        compiler_params=pltpu.CompilerParams(use_tc_tiling_on_sc=False),
    )
    def gather_kernel(a_hbm, idx_hbm, out_hbm, idx_all, a_sh, bufs, gsems, ssems):
        wid = lax.axis_index("s") * _NC + lax.axis_index("c")
        base = wid * epw

        @pl.when(lax.axis_index("s") == 0)
        def _():
            pltpu.sync_copy(a_hbm, a_sh)

        pltpu.sync_copy(idx_hbm.at[pl.ds(base, epw)], idx_all)
        plsc.subcore_barrier()

        def gather_cp(c, b):
            return pltpu.make_async_copy(
                a_sh.at[idx_all.at[pl.ds(c * chunk, chunk)]],
                bufs[b],
                gsems[b],
            )

        def store_cp(c, b):
            return pltpu.make_async_copy(
                bufs[b], out_hbm.at[pl.ds(base + c * chunk, chunk)], ssems[b]
            )

        for c in range(min(nbuf, nchunk)):
            gather_cp(c, c % nbuf).start()
        for c in range(nchunk):
            b = c % nbuf
            gather_cp(c, b).wait()
            store_cp(c, b).start()
            if c + nbuf < nchunk:
                store_cp(c, b).wait()
                gather_cp(c + nbuf, b).start()
            else:
                store_cp(c, b).wait()

    return gather_kernel


def _stage3(pack, rows_per_node):
    def body(rep_ref, b_ref, w2t_ref, b2_ref, o_ref):
        g = b_ref.shape[0]
        w = pack * b_ref.shape[1]
        z = rep_ref[...].reshape(g * rows_per_node, w).astype(jnp.float32)
        bt = jnp.concatenate([b_ref[...]] * pack, axis=1)
        z3 = z.reshape(g, rows_per_node, w) + bt[:, None, :]
        hh = z3 * 0.5 * (1.0 + lax.erf(z3 * (2.0**-0.5)))
        t = jnp.sum(hh, axis=1)
        o_ref[...] = (
            jnp.dot(t, w2t_ref[...], preferred_element_type=jnp.float32)
            + b2_ref[...]
        )

    return body


def kernel(in_features, neighbors_index, neighbors_row_splits, W1, b1, W2, b2):
    n, c = in_features.shape
    e = neighbors_index.shape[0]
    h = W1.shape[1]
    co = W2.shape[1]
    k = e // n

    a_tab, b_tab = pl.pallas_call(
        _stage1,
        out_shape=[
            jax.ShapeDtypeStruct((n, h), jnp.bfloat16),
            jax.ShapeDtypeStruct((n, h), jnp.float32),
        ],
    )(in_features, W1, b1.reshape(1, h))

    rep = _make_gather(n, h, e, 1000, 4)(a_tab, neighbors_index)

    pack = 128 // h
    rpn = k // pack
    w2t = jnp.tile(W2, (pack, 1)) * (1.0 / k)

    bn = 400
    out = pl.pallas_call(
        _stage3(pack, rpn),
        grid=(n // bn,),
        in_specs=[
            pl.BlockSpec((bn * k * h,), lambda i: (i,)),
            pl.BlockSpec((bn, h), lambda i: (i, 0)),
            pl.BlockSpec((pack * h, co), lambda i: (0, 0)),
            pl.BlockSpec((1, co), lambda i: (0, 0)),
        ],
        out_specs=pl.BlockSpec((bn, co), lambda i: (i, 0)),
        out_shape=jax.ShapeDtypeStruct((n, co), jnp.float32),
    )(rep.reshape(e * h), b_tab, w2t, b2.reshape(1, co))

    return out


# speedup vs baseline: 3.1937x; 1.7122x over previous
import functools

import jax
import jax.numpy as jnp
from jax import lax
from jax.experimental import pallas as pl
from jax.experimental.pallas import tpu as pltpu
from jax.experimental.pallas import tpu_sc as plsc

_NC = 2
_NS = 16
_NW = _NC * _NS


def _stage1(x_ref, w1_ref, b1_ref, a_ref, b_ref):
    x = x_ref[...]
    w = w1_ref[...]
    c = x.shape[1]
    a_ref[...] = jnp.dot(x, w[:c, :], preferred_element_type=jnp.float32)
    b_ref[...] = jnp.dot(x, w[c:, :], preferred_element_type=jnp.float32) + b1_ref[...]


def _make_gather(n, h, e, chunk, nbuf):
    epw = e // _NW
    nchunk = epw // chunk
    mesh = plsc.VectorSubcoreMesh(
        core_axis_name="c", subcore_axis_name="s", num_cores=_NC, num_subcores=_NS
    )

    @functools.partial(
        pl.kernel,
        out_type=jax.ShapeDtypeStruct((e, h), jnp.float32),
        mesh=mesh,
        scratch_types=[
            pltpu.VMEM((epw,), jnp.int32),
            pltpu.VMEM_SHARED((n, h), jnp.float32),
            [pltpu.VMEM((chunk, h), jnp.float32) for _ in range(nbuf)],
            [pltpu.SemaphoreType.DMA for _ in range(nbuf)],
            [pltpu.SemaphoreType.DMA for _ in range(nbuf)],
        ],
        compiler_params=pltpu.CompilerParams(use_tc_tiling_on_sc=False),
    )
    def gather_kernel(a_hbm, idx_hbm, out_hbm, idx_all, a_sh, bufs, gsems, ssems):
        wid = lax.axis_index("s") * _NC + lax.axis_index("c")
        base = wid * epw

        @pl.when(lax.axis_index("s") == 0)
        def _():
            pltpu.sync_copy(a_hbm, a_sh)

        pltpu.sync_copy(idx_hbm.at[pl.ds(base, epw)], idx_all)
        plsc.subcore_barrier()

        def gather_cp(c, b):
            return pltpu.make_async_copy(
                a_sh.at[idx_all.at[pl.ds(c * chunk, chunk)]],
                bufs[b],
                gsems[b],
            )

        def store_cp(c, b):
            return pltpu.make_async_copy(
                bufs[b], out_hbm.at[pl.ds(base + c * chunk, chunk)], ssems[b]
            )

        for c in range(min(nbuf, nchunk)):
            gather_cp(c, c % nbuf).start()
        for c in range(nchunk):
            b = c % nbuf
            gather_cp(c, b).wait()
            store_cp(c, b).start()
            if c + nbuf < nchunk:
                store_cp(c, b).wait()
                gather_cp(c + nbuf, b).start()
            else:
                store_cp(c, b).wait()

    return gather_kernel


def _stage3(pack, rows_per_node):
    def body(rep_ref, b_ref, w2t_ref, b2_ref, o_ref):
        g = b_ref.shape[0]
        w = pack * b_ref.shape[1]
        z = rep_ref[...].reshape(g * rows_per_node, w)
        bt = jnp.concatenate([b_ref[...]] * pack, axis=1)
        z3 = z.reshape(g, rows_per_node, w) + bt[:, None, :]
        hh = z3 * 0.5 * (1.0 + lax.erf(z3 * (2.0**-0.5)))
        t = jnp.sum(hh, axis=1)
        o_ref[...] = (
            jnp.dot(t, w2t_ref[...], preferred_element_type=jnp.float32)
            + b2_ref[...]
        )

    return body


def kernel(in_features, neighbors_index, neighbors_row_splits, W1, b1, W2, b2):
    n, c = in_features.shape
    e = neighbors_index.shape[0]
    h = W1.shape[1]
    co = W2.shape[1]
    k = e // n

    a_tab, b_tab = pl.pallas_call(
        _stage1,
        out_shape=[
            jax.ShapeDtypeStruct((n, h), jnp.float32),
            jax.ShapeDtypeStruct((n, h), jnp.float32),
        ],
    )(in_features, W1, b1.reshape(1, h))

    rep = _make_gather(n, h, e, 1000, 3)(a_tab, neighbors_index)

    pack = 128 // h
    rpn = k // pack
    w2t = jnp.tile(W2, (pack, 1)) * (1.0 / k)

    bn = 400
    out = pl.pallas_call(
        _stage3(pack, rpn),
        grid=(n // bn,),
        in_specs=[
            pl.BlockSpec((bn * k * h,), lambda i: (i,)),
            pl.BlockSpec((bn, h), lambda i: (i, 0)),
            pl.BlockSpec((pack * h, co), lambda i: (0, 0)),
            pl.BlockSpec((1, co), lambda i: (0, 0)),
        ],
        out_specs=pl.BlockSpec((bn, co), lambda i: (i, 0)),
        out_shape=jax.ShapeDtypeStruct((n, co), jnp.float32),
    )(rep.reshape(e * h), b_tab, w2t, b2.reshape(1, co))

    return out


# speedup vs baseline: 3.3438x; 1.0470x over previous
import functools

import jax
import jax.numpy as jnp
from jax import lax
from jax.experimental import pallas as pl
from jax.experimental.pallas import tpu as pltpu
from jax.experimental.pallas import tpu_sc as plsc

_NC = 2
_NS = 16
_NW = _NC * _NS


def _stage1(x_ref, x4_ref, w4_ref, w1b_ref, b1_ref, a4_ref, b_ref):
    a4_ref[...] = jnp.dot(
        x4_ref[...], w4_ref[...], preferred_element_type=jnp.float32
    )
    b_ref[...] = (
        jnp.dot(x_ref[...], w1b_ref[...], preferred_element_type=jnp.float32)
        + b1_ref[...]
    )


def _make_gather(n, h, e, chunk, nbuf):
    epw = e // _NW
    nchunk = epw // chunk
    mesh = plsc.VectorSubcoreMesh(
        core_axis_name="c", subcore_axis_name="s", num_cores=_NC, num_subcores=_NS
    )

    @functools.partial(
        pl.kernel,
        out_type=jax.ShapeDtypeStruct((e, h), jnp.float32),
        mesh=mesh,
        scratch_types=[
            pltpu.VMEM((epw,), jnp.int32),
            pltpu.VMEM_SHARED((n, h), jnp.float32),
            [pltpu.VMEM((chunk, h), jnp.float32) for _ in range(nbuf)],
            [pltpu.SemaphoreType.DMA for _ in range(nbuf)],
            [pltpu.SemaphoreType.DMA for _ in range(nbuf)],
        ],
        compiler_params=pltpu.CompilerParams(use_tc_tiling_on_sc=False),
    )
    def gather_kernel(a_hbm, idx_hbm, out_hbm, idx_all, a_sh, bufs, gsems, ssems):
        wid = lax.axis_index("s") * _NC + lax.axis_index("c")
        base = wid * epw

        @pl.when(lax.axis_index("s") == 0)
        def _():
            pltpu.sync_copy(a_hbm, a_sh)

        pltpu.sync_copy(idx_hbm.at[pl.ds(base, epw)], idx_all)
        plsc.subcore_barrier()

        def gather_cp(c, b):
            return pltpu.make_async_copy(
                a_sh.at[idx_all.at[pl.ds(c * chunk, chunk)]],
                bufs[b],
                gsems[b],
            )

        def store_cp(c, b):
            return pltpu.make_async_copy(
                bufs[b], out_hbm.at[pl.ds(base + c * chunk, chunk)], ssems[b]
            )

        for c in range(min(nbuf, nchunk)):
            gather_cp(c, c % nbuf).start()
        for c in range(nchunk):
            b = c % nbuf
            gather_cp(c, b).wait()
            store_cp(c, b).start()
            if c + nbuf < nchunk:
                store_cp(c, b).wait()
                gather_cp(c + nbuf, b).start()
            else:
                store_cp(c, b).wait()

    return gather_kernel


def _stage3(pack, rows_per_node):
    def body(rep_ref, b_ref, w2t_ref, b2_ref, o_ref):
        g = b_ref.shape[0]
        w = pack * b_ref.shape[1]
        z = rep_ref[...].reshape(g * rows_per_node, w)
        bt = jnp.concatenate([b_ref[...]] * pack, axis=1)
        z3 = z.reshape(g, rows_per_node, w) + bt[:, None, :]
        hh = z3 * 0.5 * (1.0 + lax.erf(z3 * (2.0**-0.5)))
        t = jnp.sum(hh, axis=1)
        o_ref[...] = (
            jnp.dot(t, w2t_ref[...], preferred_element_type=jnp.float32)
            + b2_ref[...]
        )

    return body


def kernel(in_features, neighbors_index, neighbors_row_splits, W1, b1, W2, b2):
    n, c = in_features.shape
    e = neighbors_index.shape[0]
    h = W1.shape[1]
    co = W2.shape[1]
    k = e // n

    pk = 128 // h
    w4 = jnp.kron(jnp.eye(pk, dtype=jnp.float32), W1[:c, :])
    a4_tab, b_tab = pl.pallas_call(
        _stage1,
        out_shape=[
            jax.ShapeDtypeStruct((n // pk, pk * h), jnp.float32),
            jax.ShapeDtypeStruct((n, h), jnp.float32),
        ],
    )(
        in_features,
        in_features.reshape(n // pk, pk * c),
        w4,
        W1[c:, :],
        b1.reshape(1, h),
    )

    rep = _make_gather(n, h, e, 1000, 3)(a4_tab.reshape(n, h), neighbors_index)

    pack = 128 // h
    rpn = k // pack
    w2t = jnp.tile(W2, (pack, 1)) * (1.0 / k)

    bn = 1000
    out = pl.pallas_call(
        _stage3(pack, rpn),
        grid=(n // bn,),
        in_specs=[
            pl.BlockSpec((bn * k * h,), lambda i: (i,)),
            pl.BlockSpec((bn, h), lambda i: (i, 0)),
            pl.BlockSpec((pack * h, co), lambda i: (0, 0)),
            pl.BlockSpec((1, co), lambda i: (0, 0)),
        ],
        out_specs=pl.BlockSpec((bn, co), lambda i: (i, 0)),
        out_shape=jax.ShapeDtypeStruct((n, co), jnp.float32),
    )(rep.reshape(e * h), b_tab, w2t, b2.reshape(1, co))

    return out
